# Initial kernel scaffold; baseline (speedup 1.0000x reference)
import jax
import jax.numpy as jnp
from jax.experimental import pallas as pl


def kernel(query, current_target, labels, queue, labels_queue):
    raise NotImplementedError("write your pallas kernel here")


# speedup vs baseline: 2.8369x; 2.8369x over previous
import functools

import jax
import jax.numpy as jnp
from jax import lax
from jax.experimental import pallas as pl
from jax.experimental.pallas import tpu as pltpu
from jax.experimental.pallas import tpu_sc as plsc

B = 1024
D = 128
K = 128000
TK = 5
CHUNK = 3200
NSTEPS = K // CHUNK
NEG = float("-inf")

SC_NC = 2
SC_NS = 16
NW = SC_NC * SC_NS
PERW = (B * TK) // NW
SUBG = 80


def _topk_body(t_ref, queue_ref, idx_out_ref, tn_s, rv_s, ri_s):
    i = pl.program_id(0)

    @pl.when(i == 0)
    def _init():
        t = t_ref[...]
        n = jnp.sqrt(jnp.sum(t * t, axis=1, keepdims=True))
        tn_s[...] = t / jnp.maximum(n, 1e-12)
        rv_s[...] = jnp.full((B, 16), NEG, jnp.float32)
        ri_s[...] = jnp.zeros((B, 16), jnp.int32)

    chunk = queue_ref[...]
    n = jnp.sqrt(jnp.sum(chunk * chunk, axis=1, keepdims=True))
    chunkn = chunk / jnp.maximum(n, 1e-12)
    st = lax.dot_general(tn_s[...], chunkn, (((1,), (1,)), ((), ())),
                         preferred_element_type=jnp.float32)
    lane = lax.broadcasted_iota(jnp.int32, (B, CHUNK), 1)
    base = i * CHUNK
    cv, ci = [], []
    for _ in range(TK):
        m = jnp.max(st, axis=1, keepdims=True)
        a = jnp.argmax(st, axis=1).astype(jnp.int32)
        cv.append(m)
        ci.append(a[:, None] + base)
        st = jnp.where(lane == a[:, None], NEG, st)
    rv = rv_s[...]
    ri = ri_s[...]
    pad_v = jnp.full((B, 6), NEG, jnp.float32)
    pad_i = jnp.zeros((B, 6), jnp.int32)
    V = jnp.concatenate([rv[:, :TK]] + cv + [pad_v], axis=1)
    I = jnp.concatenate([ri[:, :TK]] + ci + [pad_i], axis=1)
    lane16 = lax.broadcasted_iota(jnp.int32, (B, 16), 1)
    mv, mi = [], []
    for _ in range(TK):
        m = jnp.max(V, axis=1, keepdims=True)
        p = jnp.argmax(V, axis=1).astype(jnp.int32)
        oh = lane16 == p[:, None]
        mv.append(m)
        mi.append(jnp.sum(jnp.where(oh, I, 0), axis=1, keepdims=True))
        V = jnp.where(oh, NEG, V)
    rv_s[...] = jnp.concatenate(mv + [jnp.full((B, 11), NEG, jnp.float32)], axis=1)
    ri_s[...] = jnp.concatenate(mi + [jnp.zeros((B, 11), jnp.int32)], axis=1)

    @pl.when(i == NSTEPS - 1)
    def _fin():
        idx_out_ref[...] = ri_s[:, :8]


def _topk_indices(current_target, queue):
    return pl.pallas_call(
        _topk_body,
        grid=(NSTEPS,),
        in_specs=[
            pl.BlockSpec((B, D), lambda i: (0, 0)),
            pl.BlockSpec((CHUNK, D), lambda i: (i, 0)),
        ],
        out_specs=pl.BlockSpec((B, 8), lambda i: (0, 0)),
        out_shape=jax.ShapeDtypeStruct((B, 8), jnp.int32),
        scratch_shapes=[
            pltpu.VMEM((B, D), jnp.float32),
            pltpu.VMEM((B, 16), jnp.float32),
            pltpu.VMEM((B, 16), jnp.int32),
        ],
    )(current_target, queue)


def _sc_gather(queue, labels_queue, idx_flat):
    mesh = plsc.VectorSubcoreMesh(core_axis_name="c", subcore_axis_name="s")

    @functools.partial(
        pl.kernel,
        mesh=mesh,
        out_type=[
            jax.ShapeDtypeStruct((B * TK, D), jnp.float32),
            jax.ShapeDtypeStruct((B * TK,), jnp.int32),
        ],
        scratch_types=[
            pltpu.VMEM((SUBG,), jnp.int32),
            pltpu.VMEM((SUBG, D), jnp.float32),
            pltpu.VMEM((SUBG,), jnp.int32),
            pltpu.SemaphoreType.DMA,
        ],
    )
    def k(queue_hbm, lblq_hbm, idx_hbm, rows_out, lbl_out, idx_v, rows_v, lbl_v, sem):
        wid = lax.axis_index("s") * SC_NC + lax.axis_index("c")
        base = wid * PERW
        for g in range(PERW // SUBG):
            off = base + g * SUBG
            pltpu.sync_copy(idx_hbm.at[pl.ds(off, SUBG)], idx_v)
            pltpu.async_copy(queue_hbm.at[idx_v], rows_v, sem).wait()
            pltpu.sync_copy(rows_v, rows_out.at[pl.ds(off, SUBG)])
            pltpu.async_copy(lblq_hbm.at[idx_v], lbl_v, sem).wait()
            pltpu.sync_copy(lbl_v, lbl_out.at[pl.ds(off, SUBG)])

    return k(queue, labels_queue, idx_flat)


def _finish_body(q_ref, rows_ref, lblg_ref, labels_ref, loss_ref, pur_ref):
    q = q_ref[...]
    qn = q / jnp.maximum(jnp.sqrt(jnp.sum(q * q, axis=1, keepdims=True)), 1e-12)
    lab = labels_ref[...]
    lacc = jnp.zeros((B, 1), jnp.float32)
    macc = jnp.zeros((B, 1), jnp.float32)
    for j in range(TK):
        g = rows_ref[pl.ds(j * B, B), :]
        gn = g / jnp.maximum(jnp.sqrt(jnp.sum(g * g, axis=1, keepdims=True)), 1e-12)
        lacc = lacc + (2.0 - 2.0 * jnp.sum(qn * gn, axis=1, keepdims=True))
        lj = lblg_ref[pl.ds(j * B, B), :]
        macc = macc + (lj == lab).astype(jnp.float32)
    loss_ref[...] = (jnp.sum(lacc) / (TK * B)).reshape(1, 1)
    pur_ref[...] = (jnp.sum(macc) / (TK * B)).reshape(1, 1)


def _finish(query, rows, lblg, labels):
    return pl.pallas_call(
        _finish_body,
        out_shape=[
            jax.ShapeDtypeStruct((1, 1), jnp.float32),
            jax.ShapeDtypeStruct((1, 1), jnp.float32),
        ],
    )(query, rows, lblg, labels)


def kernel(query, current_target, labels, queue, labels_queue):
    idx8 = _topk_indices(current_target, queue)
    idx_flat = idx8[:, :TK].T.reshape(-1)
    rows, lblg = _sc_gather(queue, labels_queue, idx_flat)
    loss, pur = _finish(query, rows, lblg.reshape(-1, 1),
                        labels.reshape(-1, 1))
    return (loss.reshape(()), pur.reshape(())
            )


# speedup vs baseline: 8.1851x; 2.8852x over previous
import functools

import jax
import jax.numpy as jnp
from jax import lax
from jax.experimental import pallas as pl
from jax.experimental.pallas import tpu as pltpu
from jax.experimental.pallas import tpu_sc as plsc

B = 1024
D = 128
K = 128000
TK = 5
NCAND = 8
CHUNK = 3200
NCOL = CHUNK // 128
NSTEPS = K // CHUNK
NLVL = 4
MASK17 = (1 << 17) - 1
QSCALE = 4095.0
MAGIC = 12582912.0
KBIAS = 12224 << 17

SC_NC = 2
SC_NS = 16
NW = SC_NC * SC_NS
PERW = (B * NCAND) // NW
SUBG = 64


def _topk_body(t_ref, queue_ref, keys_ref, tn_s):
    i = pl.program_id(0)

    @pl.when(i == 0)
    def _init():
        t = t_ref[...]
        n = jnp.sqrt(jnp.sum(t * t, axis=1, keepdims=True))
        tn_s[...] = t / jnp.maximum(n, 1e-12)
        keys_ref[...] = jnp.zeros((B, NLVL * 128), jnp.float32)

    chunk = queue_ref[...]
    n = jnp.sqrt(jnp.sum(chunk * chunk, axis=1, keepdims=True))
    chunkn = chunk / jnp.maximum(n, 1e-12)
    st = lax.dot_general(tn_s[...], chunkn, (((1,), (1,)), ((), ())),
                         preferred_element_type=jnp.float32)
    qb = lax.bitcast_convert_type(st * QSCALE + MAGIC, jnp.int32)
    col = lax.broadcasted_iota(jnp.int32, (B, CHUNK), 1) + (KBIAS + i * CHUNK)
    key = lax.bitcast_convert_type((qb << 17) + col, jnp.float32)
    r = [keys_ref[:, k * 128:(k + 1) * 128] for k in range(NLVL)]
    for c in range(NCOL):
        v = key[:, c * 128:(c + 1) * 128]
        for k in range(NLVL):
            hi = jnp.maximum(r[k], v)
            v = jnp.minimum(r[k], v)
            r[k] = hi
    for k in range(NLVL):
        keys_ref[:, k * 128:(k + 1) * 128] = r[k]


def _topk_keys(current_target, queue):
    return pl.pallas_call(
        _topk_body,
        grid=(NSTEPS,),
        in_specs=[
            pl.BlockSpec((B, D), lambda i: (0, 0)),
            pl.BlockSpec((CHUNK, D), lambda i: (i, 0)),
        ],
        out_specs=pl.BlockSpec((B, NLVL * 128), lambda i: (0, 0)),
        out_shape=jax.ShapeDtypeStruct((B, NLVL * 128), jnp.float32),
        scratch_shapes=[
            pltpu.VMEM((B, D), jnp.float32),
        ],
    )(current_target, queue)


def _extract_body(keys_ref, idx_ref):
    r = [keys_ref[:, k * 128:(k + 1) * 128] for k in range(NLVL)]
    cols = []
    for _ in range(NCAND):
        m = jnp.max(r[0], axis=1, keepdims=True)
        ik = lax.bitcast_convert_type(m, jnp.int32)
        cols.append(ik & MASK17)
        f = r[0] == m
        for k in range(NLVL - 1):
            r[k] = jnp.where(f, r[k + 1], r[k])
        r[NLVL - 1] = jnp.where(f, 0.0, r[NLVL - 1])
    idx_ref[...] = jnp.concatenate(cols, axis=1)


def _extract_idx(keys):
    return pl.pallas_call(
        _extract_body,
        out_shape=jax.ShapeDtypeStruct((B, NCAND), jnp.int32),
    )(keys)


def _sc_gather(queue, labels_queue, idx_flat):
    mesh = plsc.VectorSubcoreMesh(core_axis_name="c", subcore_axis_name="s")

    @functools.partial(
        pl.kernel,
        mesh=mesh,
        out_type=[
            jax.ShapeDtypeStruct((B * NCAND, D), jnp.float32),
            jax.ShapeDtypeStruct((B * NCAND,), jnp.int32),
        ],
        scratch_types=[
            pltpu.VMEM((SUBG,), jnp.int32),
            pltpu.VMEM((SUBG, D), jnp.float32),
            pltpu.VMEM((SUBG,), jnp.int32),
            pltpu.SemaphoreType.DMA,
        ],
    )
    def k(queue_hbm, lblq_hbm, idx_hbm, rows_out, lbl_out, idx_v, rows_v, lbl_v, sem):
        wid = lax.axis_index("s") * SC_NC + lax.axis_index("c")
        base = wid * PERW
        for g in range(PERW // SUBG):
            off = base + g * SUBG
            pltpu.sync_copy(idx_hbm.at[pl.ds(off, SUBG)], idx_v)
            pltpu.async_copy(queue_hbm.at[idx_v], rows_v, sem).wait()
            pltpu.sync_copy(rows_v, rows_out.at[pl.ds(off, SUBG)])
            pltpu.async_copy(lblq_hbm.at[idx_v], lbl_v, sem).wait()
            pltpu.sync_copy(lbl_v, lbl_out.at[pl.ds(off, SUBG)])

    return k(queue, labels_queue, idx_flat)


def _finish_body(q_ref, t_ref, rows_ref, lblg_ref, labels_ref, loss_ref, pur_ref):
    q = q_ref[...]
    qn = q / jnp.maximum(jnp.sqrt(jnp.sum(q * q, axis=1, keepdims=True)), 1e-12)
    t = t_ref[...]
    tn = t / jnp.maximum(jnp.sqrt(jnp.sum(t * t, axis=1, keepdims=True)), 1e-12)
    lab = labels_ref[...]
    dts, dqs, mts = [], [], []
    for j in range(NCAND):
        g = rows_ref[pl.ds(j * B, B), :]
        gn = g / jnp.maximum(jnp.sqrt(jnp.sum(g * g, axis=1, keepdims=True)), 1e-12)
        dts.append(jnp.sum(tn * gn, axis=1, keepdims=True))
        dqs.append(2.0 - 2.0 * jnp.sum(qn * gn, axis=1, keepdims=True))
        lj = lblg_ref[pl.ds(j * B, B), :]
        mts.append((lj == lab).astype(jnp.float32))
    simt = jnp.concatenate(dts, axis=1)
    dq = jnp.concatenate(dqs, axis=1)
    mt = jnp.concatenate(mts, axis=1)
    lane = lax.broadcasted_iota(jnp.int32, (B, NCAND), 1)
    lacc = jnp.zeros((B, 1), jnp.float32)
    macc = jnp.zeros((B, 1), jnp.float32)
    for _ in range(TK):
        p = jnp.argmax(simt, axis=1).astype(jnp.int32)
        oh = lane == p[:, None]
        lacc = lacc + jnp.sum(jnp.where(oh, dq, 0.0), axis=1, keepdims=True)
        macc = macc + jnp.sum(jnp.where(oh, mt, 0.0), axis=1, keepdims=True)
        simt = jnp.where(oh, -jnp.inf, simt)
    loss_ref[...] = (jnp.sum(lacc) / (TK * B)).reshape(1, 1)
    pur_ref[...] = (jnp.sum(macc) / (TK * B)).reshape(1, 1)


def _finish(query, current_target, rows, lblg, labels):
    return pl.pallas_call(
        _finish_body,
        out_shape=[
            jax.ShapeDtypeStruct((1, 1), jnp.float32),
            jax.ShapeDtypeStruct((1, 1), jnp.float32),
        ],
    )(query, current_target, rows, lblg, labels)


def kernel(query, current_target, labels, queue, labels_queue):
    keys = _topk_keys(current_target, queue)
    idx = _extract_idx(keys)
    idx_flat = idx.T.reshape(-1)
    rows, lblg = _sc_gather(queue, labels_queue, idx_flat)
    loss, pur = _finish(query, current_target, rows,
                        lblg.reshape(-1, 1), labels.reshape(-1, 1))
    return (loss.reshape(()), pur.reshape(()))


# speedup vs baseline: 8.6587x; 1.0579x over previous
import functools

import jax
import jax.numpy as jnp
from jax import lax
from jax.experimental import pallas as pl
from jax.experimental.pallas import tpu as pltpu
from jax.experimental.pallas import tpu_sc as plsc

B = 1024
D = 128
K = 128000
TK = 5
NCAND = 8
CHUNK = 5120
NCOL = CHUNK // 128
NSTEPS = K // CHUNK
NLVL = 4
MASK17 = (1 << 17) - 1
QSCALE = 4095.0
MAGIC = 12582912.0
KBIAS = 12224 << 17

SC_NC = 2
SC_NS = 16
NW = SC_NC * SC_NS
PERW = (B * NCAND) // NW
SUBG = 64


def _topk_body(t_ref, queue_ref, keys_ref, tn_s):
    i = pl.program_id(0)

    @pl.when(i == 0)
    def _init():
        t = t_ref[...]
        n = jnp.sqrt(jnp.sum(t * t, axis=1, keepdims=True))
        tn_s[...] = t / jnp.maximum(n, 1e-12) * QSCALE
        keys_ref[...] = jnp.zeros((B, NLVL * 128), jnp.float32)

    chunk = queue_ref[...]
    n = jnp.sqrt(jnp.sum(chunk * chunk, axis=1, keepdims=True))
    chunkn = chunk / jnp.maximum(n, 1e-12)
    st = lax.dot_general(tn_s[...], chunkn, (((1,), (1,)), ((), ())),
                         preferred_element_type=jnp.float32)
    qb = lax.bitcast_convert_type(st + MAGIC, jnp.int32)
    col = lax.broadcasted_iota(jnp.int32, (B, CHUNK), 1) + (KBIAS + i * CHUNK)
    key = lax.bitcast_convert_type((qb << 17) + col, jnp.float32)
    r = [keys_ref[:, k * 128:(k + 1) * 128] for k in range(NLVL)]
    for c in range(NCOL):
        v = key[:, c * 128:(c + 1) * 128]
        for k in range(NLVL):
            hi = jnp.maximum(r[k], v)
            v = jnp.minimum(r[k], v)
            r[k] = hi
    for k in range(NLVL):
        keys_ref[:, k * 128:(k + 1) * 128] = r[k]


def _topk_keys(current_target, queue):
    return pl.pallas_call(
        _topk_body,
        grid=(NSTEPS,),
        in_specs=[
            pl.BlockSpec((B, D), lambda i: (0, 0)),
            pl.BlockSpec((CHUNK, D), lambda i: (i, 0)),
        ],
        out_specs=pl.BlockSpec((B, NLVL * 128), lambda i: (0, 0)),
        out_shape=jax.ShapeDtypeStruct((B, NLVL * 128), jnp.float32),
        scratch_shapes=[
            pltpu.VMEM((B, D), jnp.float32),
        ],
    )(current_target, queue)


def _extract_body(keys_ref, idx_ref):
    r = [keys_ref[:, k * 128:(k + 1) * 128] for k in range(NLVL)]
    cols = []
    for _ in range(NCAND):
        m = jnp.max(r[0], axis=1, keepdims=True)
        ik = lax.bitcast_convert_type(m, jnp.int32)
        cols.append(ik & MASK17)
        f = r[0] == m
        for k in range(NLVL - 1):
            r[k] = jnp.where(f, r[k + 1], r[k])
        r[NLVL - 1] = jnp.where(f, 0.0, r[NLVL - 1])
    idx_ref[...] = jnp.concatenate(cols, axis=1)


def _extract_idx(keys):
    return pl.pallas_call(
        _extract_body,
        out_shape=jax.ShapeDtypeStruct((B, NCAND), jnp.int32),
    )(keys)


def _sc_gather(queue, labels_queue, idx_flat):
    mesh = plsc.VectorSubcoreMesh(core_axis_name="c", subcore_axis_name="s")

    @functools.partial(
        pl.kernel,
        mesh=mesh,
        out_type=[
            jax.ShapeDtypeStruct((B * NCAND, D), jnp.float32),
            jax.ShapeDtypeStruct((B * NCAND,), jnp.int32),
        ],
        scratch_types=[
            pltpu.VMEM((SUBG,), jnp.int32),
            pltpu.VMEM((SUBG, D), jnp.float32),
            pltpu.VMEM((SUBG,), jnp.int32),
            pltpu.SemaphoreType.DMA,
        ],
    )
    def k(queue_hbm, lblq_hbm, idx_hbm, rows_out, lbl_out, idx_v, rows_v, lbl_v, sem):
        wid = lax.axis_index("s") * SC_NC + lax.axis_index("c")
        base = wid * PERW
        for g in range(PERW // SUBG):
            off = base + g * SUBG
            pltpu.sync_copy(idx_hbm.at[pl.ds(off, SUBG)], idx_v)
            pltpu.async_copy(queue_hbm.at[idx_v], rows_v, sem).wait()
            pltpu.sync_copy(rows_v, rows_out.at[pl.ds(off, SUBG)])
            pltpu.async_copy(lblq_hbm.at[idx_v], lbl_v, sem).wait()
            pltpu.sync_copy(lbl_v, lbl_out.at[pl.ds(off, SUBG)])

    return k(queue, labels_queue, idx_flat)


def _finish_body(q_ref, t_ref, rows_ref, lblg_ref, labels_ref, loss_ref, pur_ref):
    q = q_ref[...]
    qn = q / jnp.maximum(jnp.sqrt(jnp.sum(q * q, axis=1, keepdims=True)), 1e-12)
    t = t_ref[...]
    tn = t / jnp.maximum(jnp.sqrt(jnp.sum(t * t, axis=1, keepdims=True)), 1e-12)
    lab = labels_ref[...]
    dts, dqs, mts = [], [], []
    for j in range(NCAND):
        g = rows_ref[pl.ds(j * B, B), :]
        gn = g / jnp.maximum(jnp.sqrt(jnp.sum(g * g, axis=1, keepdims=True)), 1e-12)
        dts.append(jnp.sum(tn * gn, axis=1, keepdims=True))
        dqs.append(2.0 - 2.0 * jnp.sum(qn * gn, axis=1, keepdims=True))
        lj = lblg_ref[pl.ds(j * B, B), :]
        mts.append((lj == lab).astype(jnp.float32))
    simt = jnp.concatenate(dts, axis=1)
    dq = jnp.concatenate(dqs, axis=1)
    mt = jnp.concatenate(mts, axis=1)
    lane = lax.broadcasted_iota(jnp.int32, (B, NCAND), 1)
    lacc = jnp.zeros((B, 1), jnp.float32)
    macc = jnp.zeros((B, 1), jnp.float32)
    for _ in range(TK):
        p = jnp.argmax(simt, axis=1).astype(jnp.int32)
        oh = lane == p[:, None]
        lacc = lacc + jnp.sum(jnp.where(oh, dq, 0.0), axis=1, keepdims=True)
        macc = macc + jnp.sum(jnp.where(oh, mt, 0.0), axis=1, keepdims=True)
        simt = jnp.where(oh, -jnp.inf, simt)
    loss_ref[...] = (jnp.sum(lacc) / (TK * B)).reshape(1, 1)
    pur_ref[...] = (jnp.sum(macc) / (TK * B)).reshape(1, 1)


def _finish(query, current_target, rows, lblg, labels):
    return pl.pallas_call(
        _finish_body,
        out_shape=[
            jax.ShapeDtypeStruct((1, 1), jnp.float32),
            jax.ShapeDtypeStruct((1, 1), jnp.float32),
        ],
    )(query, current_target, rows, lblg, labels)


def kernel(query, current_target, labels, queue, labels_queue):
    keys = _topk_keys(current_target, queue)
    idx = _extract_idx(keys)
    idx_flat = idx.T.reshape(-1)
    rows, lblg = _sc_gather(queue, labels_queue, idx_flat)
    loss, pur = _finish(query, current_target, rows,
                        lblg.reshape(-1, 1), labels.reshape(-1, 1))
    return (loss.reshape(()), pur.reshape(()))


# speedup vs baseline: 9.9627x; 1.1506x over previous
import functools

import jax
import jax.numpy as jnp
from jax import lax
from jax.experimental import pallas as pl
from jax.experimental.pallas import tpu as pltpu
from jax.experimental.pallas import tpu_sc as plsc

B = 1024
D = 128
K = 128000
TK = 5
NCAND = 8
CHUNK = 5120
NCOL = CHUNK // 128
NSTEPS = K // CHUNK
NLVL = 3
MASK17 = (1 << 17) - 1
QSCALE = 4095.0
MAGIC = 12582912.0
KBIAS = 12224 << 17

SC_NC = 2
SC_NS = 16
NW = SC_NC * SC_NS
PERW = (B * NCAND) // NW
SUBG = 64


def _topk_body(t_ref, queue_ref, idx_ref, tn_s, keys_s):
    i = pl.program_id(0)

    @pl.when(i == 0)
    def _init():
        t = t_ref[...]
        n = jnp.sqrt(jnp.sum(t * t, axis=1, keepdims=True))
        tn_s[...] = t / jnp.maximum(n, 1e-12) * QSCALE
        keys_s[...] = jnp.zeros((B, NLVL * 128), jnp.float32)

    chunk = queue_ref[...]
    n = jnp.sqrt(jnp.sum(chunk * chunk, axis=1, keepdims=True))
    chunkn = chunk / jnp.maximum(n, 1e-12)
    st = lax.dot_general(tn_s[...], chunkn, (((1,), (1,)), ((), ())),
                         preferred_element_type=jnp.float32)
    qb = lax.bitcast_convert_type(st + MAGIC, jnp.int32)
    col = lax.broadcasted_iota(jnp.int32, (B, CHUNK), 1) + (KBIAS + i * CHUNK)
    key = lax.bitcast_convert_type((qb << 17) + col, jnp.float32)
    r = [keys_s[:, k * 128:(k + 1) * 128] for k in range(NLVL)]
    for c in range(NCOL):
        v = key[:, c * 128:(c + 1) * 128]
        for k in range(NLVL):
            hi = jnp.maximum(r[k], v)
            v = jnp.minimum(r[k], v)
            r[k] = hi
    for k in range(NLVL):
        keys_s[:, k * 128:(k + 1) * 128] = r[k]

    @pl.when(i == NSTEPS - 1)
    def _extract():
        s = [keys_s[:, k * 128:(k + 1) * 128] for k in range(NLVL)]
        cols = []
        for _ in range(NCAND):
            m = jnp.max(s[0], axis=1, keepdims=True)
            ik = lax.bitcast_convert_type(m, jnp.int32)
            cols.append(ik & MASK17)
            f = s[0] == m
            for k in range(NLVL - 1):
                s[k] = jnp.where(f, s[k + 1], s[k])
            s[NLVL - 1] = jnp.where(f, 0.0, s[NLVL - 1])
        idx_ref[...] = jnp.concatenate(cols, axis=1)


def _topk_indices(current_target, queue):
    return pl.pallas_call(
        _topk_body,
        grid=(NSTEPS,),
        in_specs=[
            pl.BlockSpec((B, D), lambda i: (0, 0)),
            pl.BlockSpec((CHUNK, D), lambda i: (i, 0)),
        ],
        out_specs=pl.BlockSpec((B, NCAND), lambda i: (0, 0)),
        out_shape=jax.ShapeDtypeStruct((B, NCAND), jnp.int32),
        scratch_shapes=[
            pltpu.VMEM((B, D), jnp.float32),
            pltpu.VMEM((B, NLVL * 128), jnp.float32),
        ],
    )(current_target, queue)


def _sc_gather(queue, labels_queue, idx_flat):
    mesh = plsc.VectorSubcoreMesh(core_axis_name="c", subcore_axis_name="s")

    @functools.partial(
        pl.kernel,
        mesh=mesh,
        out_type=[
            jax.ShapeDtypeStruct((B * NCAND, D), jnp.float32),
            jax.ShapeDtypeStruct((B * NCAND,), jnp.int32),
        ],
        scratch_types=[
            pltpu.VMEM((SUBG,), jnp.int32),
            pltpu.VMEM((SUBG, D), jnp.float32),
            pltpu.VMEM((SUBG,), jnp.int32),
            pltpu.SemaphoreType.DMA,
        ],
    )
    def k(queue_hbm, lblq_hbm, idx_hbm, rows_out, lbl_out, idx_v, rows_v, lbl_v, sem):
        wid = lax.axis_index("s") * SC_NC + lax.axis_index("c")
        base = wid * PERW
        for g in range(PERW // SUBG):
            off = base + g * SUBG
            pltpu.sync_copy(idx_hbm.at[pl.ds(off, SUBG)], idx_v)
            pltpu.async_copy(queue_hbm.at[idx_v], rows_v, sem).wait()
            pltpu.sync_copy(rows_v, rows_out.at[pl.ds(off, SUBG)])
            pltpu.async_copy(lblq_hbm.at[idx_v], lbl_v, sem).wait()
            pltpu.sync_copy(lbl_v, lbl_out.at[pl.ds(off, SUBG)])

    return k(queue, labels_queue, idx_flat)


def _finish_body(q_ref, t_ref, rows_ref, lblg_ref, labels_ref, loss_ref, pur_ref):
    q = q_ref[...]
    qn = q / jnp.maximum(jnp.sqrt(jnp.sum(q * q, axis=1, keepdims=True)), 1e-12)
    t = t_ref[...]
    tn = t / jnp.maximum(jnp.sqrt(jnp.sum(t * t, axis=1, keepdims=True)), 1e-12)
    lab = labels_ref[...]
    dts, dqs, mts = [], [], []
    for j in range(NCAND):
        g = rows_ref[pl.ds(j * B, B), :]
        gn = g / jnp.maximum(jnp.sqrt(jnp.sum(g * g, axis=1, keepdims=True)), 1e-12)
        dts.append(jnp.sum(tn * gn, axis=1, keepdims=True))
        dqs.append(2.0 - 2.0 * jnp.sum(qn * gn, axis=1, keepdims=True))
        lj = lblg_ref[pl.ds(j * B, B), :]
        mts.append((lj == lab).astype(jnp.float32))
    simt = jnp.concatenate(dts, axis=1)
    dq = jnp.concatenate(dqs, axis=1)
    mt = jnp.concatenate(mts, axis=1)
    lane = lax.broadcasted_iota(jnp.int32, (B, NCAND), 1)
    lacc = jnp.zeros((B, 1), jnp.float32)
    macc = jnp.zeros((B, 1), jnp.float32)
    for _ in range(TK):
        p = jnp.argmax(simt, axis=1).astype(jnp.int32)
        oh = lane == p[:, None]
        lacc = lacc + jnp.sum(jnp.where(oh, dq, 0.0), axis=1, keepdims=True)
        macc = macc + jnp.sum(jnp.where(oh, mt, 0.0), axis=1, keepdims=True)
        simt = jnp.where(oh, -jnp.inf, simt)
    loss_ref[...] = (jnp.sum(lacc) / (TK * B)).reshape(1, 1)
    pur_ref[...] = (jnp.sum(macc) / (TK * B)).reshape(1, 1)


def _finish(query, current_target, rows, lblg, labels):
    return pl.pallas_call(
        _finish_body,
        out_shape=[
            jax.ShapeDtypeStruct((1, 1), jnp.float32),
            jax.ShapeDtypeStruct((1, 1), jnp.float32),
        ],
    )(query, current_target, rows, lblg, labels)


def kernel(query, current_target, labels, queue, labels_queue):
    idx = _topk_indices(current_target, queue)
    idx_flat = idx.T.reshape(-1)
    rows, lblg = _sc_gather(queue, labels_queue, idx_flat)
    loss, pur = _finish(query, current_target, rows,
                        lblg.reshape(-1, 1), labels.reshape(-1, 1))
    return (loss.reshape(()), pur.reshape(()))


# speedup vs baseline: 11.2106x; 1.1253x over previous
import functools

import jax
import jax.numpy as jnp
from jax import lax
from jax.experimental import pallas as pl
from jax.experimental.pallas import tpu as pltpu
from jax.experimental.pallas import tpu_sc as plsc

B = 1024
D = 128
K = 128000
TK = 5
NCAND = 8
CHUNK = 5120
NCOL = CHUNK // 128
NSTEPS = K // CHUNK
NSET = 2
NLVL = 2
MASK17 = (1 << 17) - 1
QSCALE = 4095.0
MAGIC = 12582912.0
KBIAS = 12224 << 17

SC_NC = 2
SC_NS = 16
NW = SC_NC * SC_NS
PERW = (B * NCAND) // NW
SUBG = 64


def _topk_body(t_ref, queue_ref, idx_ref, tn_s, keys_s):
    i = pl.program_id(0)

    @pl.when(i == 0)
    def _init():
        t = t_ref[...]
        n = jnp.sqrt(jnp.sum(t * t, axis=1, keepdims=True))
        tn_s[...] = t / jnp.maximum(n, 1e-12) * QSCALE
        keys_s[...] = jnp.zeros((B, NSET * NLVL * 128), jnp.float32)

    chunk = queue_ref[...]
    n = jnp.sqrt(jnp.sum(chunk * chunk, axis=1, keepdims=True))
    chunkn = chunk / jnp.maximum(n, 1e-12)
    st = lax.dot_general(tn_s[...], chunkn, (((1,), (1,)), ((), ())),
                         preferred_element_type=jnp.float32)
    qb = lax.bitcast_convert_type(st + MAGIC, jnp.int32)
    col = lax.broadcasted_iota(jnp.int32, (B, CHUNK), 1) + (KBIAS + i * CHUNK)
    key = lax.bitcast_convert_type((qb << 17) + col, jnp.float32)
    r = [[keys_s[:, (s * NLVL + k) * 128:(s * NLVL + k + 1) * 128]
          for k in range(NLVL)] for s in range(NSET)]
    for c in range(NCOL):
        v = key[:, c * 128:(c + 1) * 128]
        rs = r[c % NSET]
        for k in range(NLVL):
            hi = jnp.maximum(rs[k], v)
            v = jnp.minimum(rs[k], v)
            rs[k] = hi
    for s in range(NSET):
        for k in range(NLVL):
            keys_s[:, (s * NLVL + k) * 128:(s * NLVL + k + 1) * 128] = r[s][k]

    @pl.when(i == NSTEPS - 1)
    def _extract():
        s = [[keys_s[:, (j * NLVL + k) * 128:(j * NLVL + k + 1) * 128]
              for k in range(NLVL)] for j in range(NSET)]
        cols = []
        for _ in range(NCAND):
            m = jnp.max(jnp.maximum(s[0][0], s[1][0]), axis=1, keepdims=True)
            ik = lax.bitcast_convert_type(m, jnp.int32)
            cols.append(ik & MASK17)
            for j in range(NSET):
                f = s[j][0] == m
                for k in range(NLVL - 1):
                    s[j][k] = jnp.where(f, s[j][k + 1], s[j][k])
                s[j][NLVL - 1] = jnp.where(f, 0.0, s[j][NLVL - 1])
        idx_ref[...] = jnp.concatenate(cols, axis=1)


def _topk_indices(current_target, queue):
    return pl.pallas_call(
        _topk_body,
        grid=(NSTEPS,),
        in_specs=[
            pl.BlockSpec((B, D), lambda i: (0, 0)),
            pl.BlockSpec((CHUNK, D), lambda i: (i, 0)),
        ],
        out_specs=pl.BlockSpec((B, NCAND), lambda i: (0, 0)),
        out_shape=jax.ShapeDtypeStruct((B, NCAND), jnp.int32),
        scratch_shapes=[
            pltpu.VMEM((B, D), jnp.float32),
            pltpu.VMEM((B, NSET * NLVL * 128), jnp.float32),
        ],
    )(current_target, queue)


def _sc_gather(queue, labels_queue, idx_flat):
    mesh = plsc.VectorSubcoreMesh(core_axis_name="c", subcore_axis_name="s")

    @functools.partial(
        pl.kernel,
        mesh=mesh,
        out_type=[
            jax.ShapeDtypeStruct((B * NCAND, D), jnp.float32),
            jax.ShapeDtypeStruct((B * NCAND,), jnp.int32),
        ],
        scratch_types=[
            pltpu.VMEM((SUBG,), jnp.int32),
            pltpu.VMEM((SUBG, D), jnp.float32),
            pltpu.VMEM((SUBG,), jnp.int32),
            pltpu.SemaphoreType.DMA,
        ],
    )
    def k(queue_hbm, lblq_hbm, idx_hbm, rows_out, lbl_out, idx_v, rows_v, lbl_v, sem):
        wid = lax.axis_index("s") * SC_NC + lax.axis_index("c")
        base = wid * PERW
        for g in range(PERW // SUBG):
            off = base + g * SUBG
            pltpu.sync_copy(idx_hbm.at[pl.ds(off, SUBG)], idx_v)
            pltpu.async_copy(queue_hbm.at[idx_v], rows_v, sem).wait()
            pltpu.sync_copy(rows_v, rows_out.at[pl.ds(off, SUBG)])
            pltpu.async_copy(lblq_hbm.at[idx_v], lbl_v, sem).wait()
            pltpu.sync_copy(lbl_v, lbl_out.at[pl.ds(off, SUBG)])

    return k(queue, labels_queue, idx_flat)


def _finish_body(q_ref, t_ref, rows_ref, lblg_ref, labels_ref, loss_ref, pur_ref):
    q = q_ref[...]
    qn = q / jnp.maximum(jnp.sqrt(jnp.sum(q * q, axis=1, keepdims=True)), 1e-12)
    t = t_ref[...]
    tn = t / jnp.maximum(jnp.sqrt(jnp.sum(t * t, axis=1, keepdims=True)), 1e-12)
    lab = labels_ref[...]
    dts, dqs, mts = [], [], []
    for j in range(NCAND):
        g = rows_ref[pl.ds(j * B, B), :]
        gn = g / jnp.maximum(jnp.sqrt(jnp.sum(g * g, axis=1, keepdims=True)), 1e-12)
        dts.append(jnp.sum(tn * gn, axis=1, keepdims=True))
        dqs.append(2.0 - 2.0 * jnp.sum(qn * gn, axis=1, keepdims=True))
        lj = lblg_ref[pl.ds(j * B, B), :]
        mts.append((lj == lab).astype(jnp.float32))
    simt = jnp.concatenate(dts, axis=1)
    dq = jnp.concatenate(dqs, axis=1)
    mt = jnp.concatenate(mts, axis=1)
    lane = lax.broadcasted_iota(jnp.int32, (B, NCAND), 1)
    lacc = jnp.zeros((B, 1), jnp.float32)
    macc = jnp.zeros((B, 1), jnp.float32)
    for _ in range(TK):
        p = jnp.argmax(simt, axis=1).astype(jnp.int32)
        oh = lane == p[:, None]
        lacc = lacc + jnp.sum(jnp.where(oh, dq, 0.0), axis=1, keepdims=True)
        macc = macc + jnp.sum(jnp.where(oh, mt, 0.0), axis=1, keepdims=True)
        simt = jnp.where(oh, -jnp.inf, simt)
    loss_ref[...] = (jnp.sum(lacc) / (TK * B)).reshape(1, 1)
    pur_ref[...] = (jnp.sum(macc) / (TK * B)).reshape(1, 1)


def _finish(query, current_target, rows, lblg, labels):
    return pl.pallas_call(
        _finish_body,
        out_shape=[
            jax.ShapeDtypeStruct((1, 1), jnp.float32),
            jax.ShapeDtypeStruct((1, 1), jnp.float32),
        ],
    )(query, current_target, rows, lblg, labels)


def kernel(query, current_target, labels, queue, labels_queue):
    idx = _topk_indices(current_target, queue)
    idx_flat = idx.T.reshape(-1)
    rows, lblg = _sc_gather(queue, labels_queue, idx_flat)
    loss, pur = _finish(query, current_target, rows,
                        lblg.reshape(-1, 1), labels.reshape(-1, 1))
    return (loss.reshape(()), pur.reshape(()))


# speedup vs baseline: 11.2985x; 1.0078x over previous
import functools

import jax
import jax.numpy as jnp
from jax import lax
from jax.experimental import pallas as pl
from jax.experimental.pallas import tpu as pltpu
from jax.experimental.pallas import tpu_sc as plsc

B = 1024
D = 128
K = 128000
TK = 5
NCAND = 8
CHUNK = 5120
NCOL = CHUNK // 128
NSTEPS = K // CHUNK
NSET = 2
NLVL = 2
MASK17 = (1 << 17) - 1
QSCALE = 4095.0
MAGIC = 12582912.0
KBIAS = 12224 << 17

SC_NC = 2
SC_NS = 16
NW = SC_NC * SC_NS
PERW = (B * NCAND) // NW
SUBG = 64


def _topk_body(t_ref, queue_ref, idx_ref, tn_s, keys_s):
    i = pl.program_id(0)

    @pl.when(i == 0)
    def _init():
        t = t_ref[...]
        n = jnp.sqrt(jnp.sum(t * t, axis=1, keepdims=True))
        tn_s[...] = t / jnp.maximum(n, 1e-12) * QSCALE
        keys_s[...] = jnp.zeros((B, NSET * NLVL * 128), jnp.float32)

    chunk = queue_ref[...]
    n = jnp.sqrt(jnp.sum(chunk * chunk, axis=1, keepdims=True))
    chunkn = chunk / jnp.maximum(n, 1e-12)
    st = lax.dot_general(tn_s[...], chunkn, (((1,), (1,)), ((), ())),
                         preferred_element_type=jnp.float32)
    qb = lax.bitcast_convert_type(st + MAGIC, jnp.int32)
    col = lax.broadcasted_iota(jnp.int32, (B, CHUNK), 1) + (KBIAS + i * CHUNK)
    key = lax.bitcast_convert_type((qb << 17) + col, jnp.float32)
    r = [[keys_s[:, (s * NLVL + k) * 128:(s * NLVL + k + 1) * 128]
          for k in range(NLVL)] for s in range(NSET)]
    for c in range(NCOL):
        v = key[:, c * 128:(c + 1) * 128]
        rs = r[c % NSET]
        for k in range(NLVL):
            hi = jnp.maximum(rs[k], v)
            v = jnp.minimum(rs[k], v)
            rs[k] = hi
    for s in range(NSET):
        for k in range(NLVL):
            keys_s[:, (s * NLVL + k) * 128:(s * NLVL + k + 1) * 128] = r[s][k]

    @pl.when(i == NSTEPS - 1)
    def _extract():
        s = [[keys_s[:, (j * NLVL + k) * 128:(j * NLVL + k + 1) * 128]
              for k in range(NLVL)] for j in range(NSET)]
        cols = []
        for _ in range(NCAND):
            m = jnp.max(jnp.maximum(s[0][0], s[1][0]), axis=1, keepdims=True)
            ik = lax.bitcast_convert_type(m, jnp.int32)
            cols.append(ik & MASK17)
            for j in range(NSET):
                f = s[j][0] == m
                for k in range(NLVL - 1):
                    s[j][k] = jnp.where(f, s[j][k + 1], s[j][k])
                s[j][NLVL - 1] = jnp.where(f, 0.0, s[j][NLVL - 1])
        idx_ref[...] = jnp.concatenate(cols, axis=1)


def _topk_indices(current_target, queue):
    return pl.pallas_call(
        _topk_body,
        grid=(NSTEPS,),
        in_specs=[
            pl.BlockSpec((B, D), lambda i: (0, 0)),
            pl.BlockSpec((CHUNK, D), lambda i: (i, 0)),
        ],
        out_specs=pl.BlockSpec((B, NCAND), lambda i: (0, 0)),
        out_shape=jax.ShapeDtypeStruct((B, NCAND), jnp.int32),
        scratch_shapes=[
            pltpu.VMEM((B, D), jnp.float32),
            pltpu.VMEM((B, NSET * NLVL * 128), jnp.float32),
        ],
    )(current_target, queue)


def _sc_gather(queue, labels_queue, idx_flat):
    mesh = plsc.VectorSubcoreMesh(core_axis_name="c", subcore_axis_name="s")

    @functools.partial(
        pl.kernel,
        mesh=mesh,
        out_type=[
            jax.ShapeDtypeStruct((B * NCAND, D), jnp.float32),
            jax.ShapeDtypeStruct((B * NCAND,), jnp.int32),
        ],
        scratch_types=[
            pltpu.VMEM((SUBG,), jnp.int32),
            pltpu.VMEM((SUBG, D), jnp.float32),
            pltpu.VMEM((SUBG,), jnp.int32),
            pltpu.SemaphoreType.DMA,
            pltpu.SemaphoreType.DMA,
        ],
    )
    def k(queue_hbm, lblq_hbm, idx_hbm, rows_out, lbl_out, idx_v, rows_v, lbl_v, sem, sem2):
        wid = lax.axis_index("s") * SC_NC + lax.axis_index("c")
        base = wid * PERW
        for g in range(PERW // SUBG):
            off = base + g * SUBG
            pltpu.sync_copy(idx_hbm.at[pl.ds(off, SUBG)], idx_v)
            crows = pltpu.async_copy(queue_hbm.at[idx_v], rows_v, sem)
            clbl = pltpu.async_copy(lblq_hbm.at[idx_v], lbl_v, sem2)
            crows.wait()
            pltpu.sync_copy(rows_v, rows_out.at[pl.ds(off, SUBG)])
            clbl.wait()
            pltpu.sync_copy(lbl_v, lbl_out.at[pl.ds(off, SUBG)])

    return k(queue, labels_queue, idx_flat)


def _finish_body(q_ref, t_ref, rows_ref, lblg_ref, labels_ref, loss_ref, pur_ref):
    q = q_ref[...]
    qn = q / jnp.maximum(jnp.sqrt(jnp.sum(q * q, axis=1, keepdims=True)), 1e-12)
    t = t_ref[...]
    tn = t / jnp.maximum(jnp.sqrt(jnp.sum(t * t, axis=1, keepdims=True)), 1e-12)
    lab = labels_ref[...]
    dts, dqs, mts = [], [], []
    for j in range(NCAND):
        g = rows_ref[pl.ds(j * B, B), :]
        gn = g / jnp.maximum(jnp.sqrt(jnp.sum(g * g, axis=1, keepdims=True)), 1e-12)
        dts.append(jnp.sum(tn * gn, axis=1, keepdims=True))
        dqs.append(2.0 - 2.0 * jnp.sum(qn * gn, axis=1, keepdims=True))
        lj = lblg_ref[pl.ds(j * B, B), :]
        mts.append((lj == lab).astype(jnp.float32))
    simt = jnp.concatenate(dts, axis=1)
    dq = jnp.concatenate(dqs, axis=1)
    mt = jnp.concatenate(mts, axis=1)
    lane = lax.broadcasted_iota(jnp.int32, (B, NCAND), 1)
    lacc = jnp.zeros((B, 1), jnp.float32)
    macc = jnp.zeros((B, 1), jnp.float32)
    for _ in range(TK):
        p = jnp.argmax(simt, axis=1).astype(jnp.int32)
        oh = lane == p[:, None]
        lacc = lacc + jnp.sum(jnp.where(oh, dq, 0.0), axis=1, keepdims=True)
        macc = macc + jnp.sum(jnp.where(oh, mt, 0.0), axis=1, keepdims=True)
        simt = jnp.where(oh, -jnp.inf, simt)
    loss_ref[...] = (jnp.sum(lacc) / (TK * B)).reshape(1, 1)
    pur_ref[...] = (jnp.sum(macc) / (TK * B)).reshape(1, 1)


def _finish(query, current_target, rows, lblg, labels):
    return pl.pallas_call(
        _finish_body,
        out_shape=[
            jax.ShapeDtypeStruct((1, 1), jnp.float32),
            jax.ShapeDtypeStruct((1, 1), jnp.float32),
        ],
    )(query, current_target, rows, lblg, labels)


def kernel(query, current_target, labels, queue, labels_queue):
    idx = _topk_indices(current_target, queue)
    idx_flat = idx.T.reshape(-1)
    rows, lblg = _sc_gather(queue, labels_queue, idx_flat)
    loss, pur = _finish(query, current_target, rows,
                        lblg.reshape(-1, 1), labels.reshape(-1, 1))
    return (loss.reshape(()), pur.reshape(()))


# speedup vs baseline: 11.4847x; 1.0165x over previous
import functools

import jax
import jax.numpy as jnp
from jax import lax
from jax.experimental import pallas as pl
from jax.experimental.pallas import tpu as pltpu
from jax.experimental.pallas import tpu_sc as plsc

B = 1024
D = 128
K = 128000
TK = 5
NCAND = 8
CHUNK = 5120
NCOL = CHUNK // 128
NSTEPS = K // CHUNK
NSET = 2
NLVL = 2
MASK17 = (1 << 17) - 1
QSCALE = 4095.0
MAGIC = 12582912.0
KBIAS = 12224 << 17

SC_NC = 2
SC_NS = 16
NW = SC_NC * SC_NS
PERW = (B * NCAND) // NW
SUBG = 128


def _topk_body(t_ref, queue_ref, idx_ref, tn_s, keys_s):
    i = pl.program_id(0)

    @pl.when(i == 0)
    def _init():
        t = t_ref[...]
        n = jnp.sqrt(jnp.sum(t * t, axis=1, keepdims=True))
        tn_s[...] = (t / jnp.maximum(n, 1e-12) * QSCALE).astype(jnp.bfloat16)
        keys_s[...] = jnp.zeros((B, NSET * NLVL * 128), jnp.float32)

    chunk = queue_ref[...]
    n = jnp.sqrt(jnp.sum(chunk * chunk, axis=1, keepdims=True))
    chunkn = (chunk / jnp.maximum(n, 1e-12)).astype(jnp.bfloat16)
    st = lax.dot_general(tn_s[...], chunkn, (((1,), (1,)), ((), ())),
                         preferred_element_type=jnp.float32)
    qb = lax.bitcast_convert_type(st + MAGIC, jnp.int32)
    col = lax.broadcasted_iota(jnp.int32, (B, CHUNK), 1) + (KBIAS + i * CHUNK)
    key = lax.bitcast_convert_type((qb << 17) + col, jnp.float32)
    r = [[keys_s[:, (s * NLVL + k) * 128:(s * NLVL + k + 1) * 128]
          for k in range(NLVL)] for s in range(NSET)]
    for c in range(NCOL):
        v = key[:, c * 128:(c + 1) * 128]
        rs = r[c % NSET]
        for k in range(NLVL):
            hi = jnp.maximum(rs[k], v)
            v = jnp.minimum(rs[k], v)
            rs[k] = hi
    for s in range(NSET):
        for k in range(NLVL):
            keys_s[:, (s * NLVL + k) * 128:(s * NLVL + k + 1) * 128] = r[s][k]

    @pl.when(i == NSTEPS - 1)
    def _extract():
        s = [[keys_s[:, (j * NLVL + k) * 128:(j * NLVL + k + 1) * 128]
              for k in range(NLVL)] for j in range(NSET)]
        cols = []
        for _ in range(NCAND):
            m = jnp.max(jnp.maximum(s[0][0], s[1][0]), axis=1, keepdims=True)
            ik = lax.bitcast_convert_type(m, jnp.int32)
            cols.append(ik & MASK17)
            for j in range(NSET):
                f = s[j][0] == m
                for k in range(NLVL - 1):
                    s[j][k] = jnp.where(f, s[j][k + 1], s[j][k])
                s[j][NLVL - 1] = jnp.where(f, 0.0, s[j][NLVL - 1])
        idx_ref[...] = jnp.concatenate(cols, axis=1)


def _topk_indices(current_target, queue):
    return pl.pallas_call(
        _topk_body,
        grid=(NSTEPS,),
        in_specs=[
            pl.BlockSpec((B, D), lambda i: (0, 0)),
            pl.BlockSpec((CHUNK, D), lambda i: (i, 0)),
        ],
        out_specs=pl.BlockSpec((B, NCAND), lambda i: (0, 0)),
        out_shape=jax.ShapeDtypeStruct((B, NCAND), jnp.int32),
        scratch_shapes=[
            pltpu.VMEM((B, D), jnp.bfloat16),
            pltpu.VMEM((B, NSET * NLVL * 128), jnp.float32),
        ],
    )(current_target, queue)


def _sc_gather(queue, labels_queue, idx_flat):
    mesh = plsc.VectorSubcoreMesh(core_axis_name="c", subcore_axis_name="s")

    @functools.partial(
        pl.kernel,
        mesh=mesh,
        out_type=[
            jax.ShapeDtypeStruct((B * NCAND, D), jnp.float32),
            jax.ShapeDtypeStruct((B * NCAND,), jnp.int32),
        ],
        scratch_types=[
            pltpu.VMEM((PERW,), jnp.int32),
            pltpu.VMEM((PERW, D), jnp.float32),
            pltpu.VMEM((PERW,), jnp.int32),
            pltpu.SemaphoreType.DMA,
            pltpu.SemaphoreType.DMA,
        ],
    )
    def k(queue_hbm, lblq_hbm, idx_hbm, rows_out, lbl_out, idx_v, rows_v, lbl_v, sem, sem2):
        wid = lax.axis_index("s") * SC_NC + lax.axis_index("c")
        base = wid * PERW
        pltpu.sync_copy(idx_hbm.at[pl.ds(base, PERW)], idx_v)
        copies = []
        for g in range(PERW // SUBG):
            off = g * SUBG
            copies.append(pltpu.async_copy(
                queue_hbm.at[idx_v.at[pl.ds(off, SUBG)]],
                rows_v.at[pl.ds(off, SUBG)], sem))
            copies.append(pltpu.async_copy(
                lblq_hbm.at[idx_v.at[pl.ds(off, SUBG)]],
                lbl_v.at[pl.ds(off, SUBG)], sem2))
        for c in copies:
            c.wait()
        pltpu.sync_copy(rows_v, rows_out.at[pl.ds(base, PERW)])
        pltpu.sync_copy(lbl_v, lbl_out.at[pl.ds(base, PERW)])

    return k(queue, labels_queue, idx_flat)


def _finish_body(q_ref, t_ref, rows_ref, lblg_ref, labels_ref, loss_ref, pur_ref):
    q = q_ref[...]
    qn = q / jnp.maximum(jnp.sqrt(jnp.sum(q * q, axis=1, keepdims=True)), 1e-12)
    t = t_ref[...]
    tn = t / jnp.maximum(jnp.sqrt(jnp.sum(t * t, axis=1, keepdims=True)), 1e-12)
    lab = labels_ref[...]
    dts, dqs, mts = [], [], []
    for j in range(NCAND):
        g = rows_ref[pl.ds(j * B, B), :]
        gn = g / jnp.maximum(jnp.sqrt(jnp.sum(g * g, axis=1, keepdims=True)), 1e-12)
        dts.append(jnp.sum(tn * gn, axis=1, keepdims=True))
        dqs.append(2.0 - 2.0 * jnp.sum(qn * gn, axis=1, keepdims=True))
        lj = lblg_ref[pl.ds(j * B, B), :]
        mts.append((lj == lab).astype(jnp.float32))
    simt = jnp.concatenate(dts, axis=1)
    dq = jnp.concatenate(dqs, axis=1)
    mt = jnp.concatenate(mts, axis=1)
    lane = lax.broadcasted_iota(jnp.int32, (B, NCAND), 1)
    lacc = jnp.zeros((B, 1), jnp.float32)
    macc = jnp.zeros((B, 1), jnp.float32)
    for _ in range(TK):
        p = jnp.argmax(simt, axis=1).astype(jnp.int32)
        oh = lane == p[:, None]
        lacc = lacc + jnp.sum(jnp.where(oh, dq, 0.0), axis=1, keepdims=True)
        macc = macc + jnp.sum(jnp.where(oh, mt, 0.0), axis=1, keepdims=True)
        simt = jnp.where(oh, -jnp.inf, simt)
    loss_ref[...] = (jnp.sum(lacc) / (TK * B)).reshape(1, 1)
    pur_ref[...] = (jnp.sum(macc) / (TK * B)).reshape(1, 1)


def _finish(query, current_target, rows, lblg, labels):
    return pl.pallas_call(
        _finish_body,
        out_shape=[
            jax.ShapeDtypeStruct((1, 1), jnp.float32),
            jax.ShapeDtypeStruct((1, 1), jnp.float32),
        ],
    )(query, current_target, rows, lblg, labels)


def kernel(query, current_target, labels, queue, labels_queue):
    idx = _topk_indices(current_target, queue)
    idx_flat = idx.T.reshape(-1)
    rows, lblg = _sc_gather(queue, labels_queue, idx_flat)
    loss, pur = _finish(query, current_target, rows,
                        lblg.reshape(-1, 1), labels.reshape(-1, 1))
    return (loss.reshape(()), pur.reshape(()))


# speedup vs baseline: 11.5415x; 1.0049x over previous
import functools

import jax
import jax.numpy as jnp
from jax import lax
from jax.experimental import pallas as pl
from jax.experimental.pallas import tpu as pltpu
from jax.experimental.pallas import tpu_sc as plsc

B = 1024
D = 128
K = 128000
TK = 5
NCAND = 8
CHUNK = 5120
NCOL = CHUNK // 128
NSTEPS = K // CHUNK
NSET = 2
NLVL = 2
MASK17 = (1 << 17) - 1
QSCALE = 4095.0
MAGIC = 12582912.0
KBIAS = 12224 << 17

SC_NC = 2
SC_NS = 16
NW = SC_NC * SC_NS
PERW = (B * NCAND) // NW
SUBG = 128


def _topk_body(t_ref, queue_ref, idx_ref, tn_s, keys_s):
    i = pl.program_id(0)

    @pl.when(i == 0)
    def _init():
        t = t_ref[...]
        n = jnp.sqrt(jnp.sum(t * t, axis=1, keepdims=True))
        tn_s[...] = t / jnp.maximum(n, 1e-12) * QSCALE
        keys_s[...] = jnp.zeros((B, NSET * NLVL * 128), jnp.float32)

    chunk = queue_ref[...]
    n = jnp.sqrt(jnp.sum(chunk * chunk, axis=1, keepdims=True))
    chunkn = chunk / jnp.maximum(n, 1e-12)
    st = lax.dot_general(tn_s[...], chunkn, (((1,), (1,)), ((), ())),
                         preferred_element_type=jnp.float32)
    qb = lax.bitcast_convert_type(st + MAGIC, jnp.int32)
    col = lax.broadcasted_iota(jnp.int32, (B, CHUNK), 1) + (KBIAS + i * CHUNK)
    key = lax.bitcast_convert_type((qb << 17) + col, jnp.float32)
    r = [[keys_s[:, (s * NLVL + k) * 128:(s * NLVL + k + 1) * 128]
          for k in range(NLVL)] for s in range(NSET)]
    for c in range(NCOL):
        v = key[:, c * 128:(c + 1) * 128]
        rs = r[c % NSET]
        for k in range(NLVL):
            hi = jnp.maximum(rs[k], v)
            v = jnp.minimum(rs[k], v)
            rs[k] = hi
    for s in range(NSET):
        for k in range(NLVL):
            keys_s[:, (s * NLVL + k) * 128:(s * NLVL + k + 1) * 128] = r[s][k]

    @pl.when(i == NSTEPS - 1)
    def _extract():
        s = [[keys_s[:, (j * NLVL + k) * 128:(j * NLVL + k + 1) * 128]
              for k in range(NLVL)] for j in range(NSET)]
        cols = []
        for _ in range(NCAND):
            m = jnp.max(jnp.maximum(s[0][0], s[1][0]), axis=1, keepdims=True)
            ik = lax.bitcast_convert_type(m, jnp.int32)
            cols.append(ik & MASK17)
            for j in range(NSET):
                f = s[j][0] == m
                for k in range(NLVL - 1):
                    s[j][k] = jnp.where(f, s[j][k + 1], s[j][k])
                s[j][NLVL - 1] = jnp.where(f, 0.0, s[j][NLVL - 1])
        idx_ref[...] = jnp.concatenate(cols, axis=1)


def _topk_indices(current_target, queue):
    return pl.pallas_call(
        _topk_body,
        grid=(NSTEPS,),
        in_specs=[
            pl.BlockSpec((B, D), lambda i: (0, 0)),
            pl.BlockSpec((CHUNK, D), lambda i: (i, 0)),
        ],
        out_specs=pl.BlockSpec((B, NCAND), lambda i: (0, 0)),
        out_shape=jax.ShapeDtypeStruct((B, NCAND), jnp.int32),
        scratch_shapes=[
            pltpu.VMEM((B, D), jnp.float32),
            pltpu.VMEM((B, NSET * NLVL * 128), jnp.float32),
        ],
    )(current_target, queue)


def _sc_gather(queue, labels_queue, idx_flat):
    mesh = plsc.VectorSubcoreMesh(core_axis_name="c", subcore_axis_name="s")

    @functools.partial(
        pl.kernel,
        mesh=mesh,
        out_type=[
            jax.ShapeDtypeStruct((B * NCAND, D), jnp.float32),
            jax.ShapeDtypeStruct((B * NCAND,), jnp.int32),
        ],
        scratch_types=[
            pltpu.VMEM((PERW,), jnp.int32),
            pltpu.VMEM((PERW, D), jnp.float32),
            pltpu.VMEM((PERW,), jnp.int32),
            pltpu.SemaphoreType.DMA,
            pltpu.SemaphoreType.DMA,
        ],
    )
    def k(queue_hbm, lblq_hbm, idx_hbm, rows_out, lbl_out, idx_v, rows_v, lbl_v, sem, sem2):
        wid = lax.axis_index("s") * SC_NC + lax.axis_index("c")
        base = wid * PERW
        pltpu.sync_copy(idx_hbm.at[pl.ds(base, PERW)], idx_v)
        copies = []
        for g in range(PERW // SUBG):
            off = g * SUBG
            copies.append(pltpu.async_copy(
                queue_hbm.at[idx_v.at[pl.ds(off, SUBG)]],
                rows_v.at[pl.ds(off, SUBG)], sem))
            copies.append(pltpu.async_copy(
                lblq_hbm.at[idx_v.at[pl.ds(off, SUBG)]],
                lbl_v.at[pl.ds(off, SUBG)], sem2))
        for c in copies:
            c.wait()
        pltpu.sync_copy(rows_v, rows_out.at[pl.ds(base, PERW)])
        pltpu.sync_copy(lbl_v, lbl_out.at[pl.ds(base, PERW)])

    return k(queue, labels_queue, idx_flat)


def _finish_body(q_ref, t_ref, rows_ref, lblg_ref, labels_ref, loss_ref, pur_ref):
    q = q_ref[...]
    qn = q / jnp.maximum(jnp.sqrt(jnp.sum(q * q, axis=1, keepdims=True)), 1e-12)
    t = t_ref[...]
    tn = t / jnp.maximum(jnp.sqrt(jnp.sum(t * t, axis=1, keepdims=True)), 1e-12)
    lab = labels_ref[...]
    dts, dqs, mts = [], [], []
    for j in range(NCAND):
        g = rows_ref[pl.ds(j * B, B), :]
        gn = g / jnp.maximum(jnp.sqrt(jnp.sum(g * g, axis=1, keepdims=True)), 1e-12)
        dts.append(jnp.sum(tn * gn, axis=1, keepdims=True))
        dqs.append(2.0 - 2.0 * jnp.sum(qn * gn, axis=1, keepdims=True))
        lj = lblg_ref[pl.ds(j * B, B), :]
        mts.append((lj == lab).astype(jnp.float32))
    simt = jnp.concatenate(dts, axis=1)
    dq = jnp.concatenate(dqs, axis=1)
    mt = jnp.concatenate(mts, axis=1)
    lane = lax.broadcasted_iota(jnp.int32, (B, NCAND), 1)
    lacc = jnp.zeros((B, 1), jnp.float32)
    macc = jnp.zeros((B, 1), jnp.float32)
    for _ in range(TK):
        p = jnp.argmax(simt, axis=1).astype(jnp.int32)
        oh = lane == p[:, None]
        lacc = lacc + jnp.sum(jnp.where(oh, dq, 0.0), axis=1, keepdims=True)
        macc = macc + jnp.sum(jnp.where(oh, mt, 0.0), axis=1, keepdims=True)
        simt = jnp.where(oh, -jnp.inf, simt)
    loss_ref[...] = (jnp.sum(lacc) / (TK * B)).reshape(1, 1)
    pur_ref[...] = (jnp.sum(macc) / (TK * B)).reshape(1, 1)


def _finish(query, current_target, rows, lblg, labels):
    return pl.pallas_call(
        _finish_body,
        out_shape=[
            jax.ShapeDtypeStruct((1, 1), jnp.float32),
            jax.ShapeDtypeStruct((1, 1), jnp.float32),
        ],
    )(query, current_target, rows, lblg, labels)


def kernel(query, current_target, labels, queue, labels_queue):
    idx = _topk_indices(current_target, queue)
    idx_flat = idx.T.reshape(-1)
    rows, lblg = _sc_gather(queue, labels_queue, idx_flat)
    loss, pur = _finish(query, current_target, rows,
                        lblg.reshape(-1, 1), labels.reshape(-1, 1))
    return (loss.reshape(()), pur.reshape(()))
